# Initial kernel scaffold; baseline (speedup 1.0000x reference)
import jax
import jax.numpy as jnp
from jax.experimental import pallas as pl


def kernel(inputs, embedding, cluster_size):
    raise NotImplementedError("write your pallas kernel here")


# speedup vs baseline: 1.3107x; 1.3107x over previous
import functools

import jax
import jax.numpy as jnp
from jax import lax
from jax.experimental import pallas as pl
from jax.experimental.pallas import tpu as pltpu
from jax.experimental.pallas import tpu_sc as plsc

N = 16384
D = 32
K = 8192
BN = 256
COMMIT = 0.25
EPS = 1e-05

SC_CORES = 2
SC_SUBCORES = 16
NW = SC_CORES * SC_SUBCORES
B_PER_W = N // NW


def _dist_argmin_body(x_ref, e_ref, sqx_ref, sqe_ref, idx_ref, et_ref):
    i = pl.program_id(0)
    e = e_ref[...]
    x = x_ref[...]
    s = jnp.dot(x.astype(jnp.bfloat16), e.astype(jnp.bfloat16),
                preferred_element_type=jnp.float32)
    dist = (sqx_ref[...] + sqe_ref[...]) - 2.0 * s
    m = jnp.min(dist, axis=1, keepdims=True)
    lanes = lax.broadcasted_iota(jnp.int32, (BN, K), 1)
    idx_ref[...] = jnp.min(jnp.where(dist == m, lanes, K), axis=1)

    @pl.when(i == 0)
    def _():
        et_ref[...] = e.T


def _dist_argmin(x, e, sqx, sqe):
    return pl.pallas_call(
        _dist_argmin_body,
        grid=(N // BN,),
        in_specs=[
            pl.BlockSpec((BN, D), lambda i: (i, 0)),
            pl.BlockSpec((D, K), lambda i: (0, 0)),
            pl.BlockSpec((BN, 1), lambda i: (i, 0)),
            pl.BlockSpec((1, K), lambda i: (0, 0)),
        ],
        out_specs=[
            pl.BlockSpec((BN,), lambda i: (i,)),
            pl.BlockSpec((K, D), lambda i: (0, 0)),
        ],
        out_shape=[
            jax.ShapeDtypeStruct((N,), jnp.int32),
            jax.ShapeDtypeStruct((K, D), jnp.float32),
        ],
    )(x, e, sqx, sqe)


def _gather_body(table_hbm, idx_hbm, out_hbm, idx_v, rows_v, sem):
    wid = lax.axis_index("s") * SC_CORES + lax.axis_index("c")
    base = wid * B_PER_W
    pltpu.sync_copy(idx_hbm.at[pl.ds(base, B_PER_W)], idx_v)
    pltpu.async_copy(table_hbm.at[idx_v], rows_v, sem).wait()
    pltpu.sync_copy(rows_v, out_hbm.at[pl.ds(base, B_PER_W)])


@functools.cache
def _get_sc_gather():
    return pl.kernel(
        _gather_body,
        mesh=plsc.VectorSubcoreMesh(
            core_axis_name="c", subcore_axis_name="s",
            num_cores=SC_CORES, num_subcores=SC_SUBCORES),
        out_type=jax.ShapeDtypeStruct((N, D), jnp.float32),
        scratch_types=[
            pltpu.VMEM((B_PER_W,), jnp.int32),
            pltpu.VMEM((B_PER_W, D), jnp.float32),
            pltpu.SemaphoreType.DMA,
        ],
        compiler_params=pltpu.CompilerParams(use_tc_tiling_on_sc=False),
    )


def _loss_stats_body(x_ref, q_ref, cs_ref, qst_ref, el_ref, cl_ref, vl_ref,
                     pp_ref, ur_ref):
    x = x_ref[...]
    q = q_ref[...]
    qst_ref[...] = x + (q - x)
    mse = jnp.mean((q - x) ** 2)
    commit = COMMIT * mse
    el_ref[...] = jnp.broadcast_to(mse, (1, 1))
    cl_ref[...] = jnp.broadcast_to(commit, (1, 1))
    vl_ref[...] = jnp.broadcast_to(mse + commit, (1, 1))
    cs = cs_ref[...]
    p = cs / (jnp.sum(cs) + EPS)
    perp = jnp.exp(-jnp.sum(p * jnp.log(p + EPS)))
    pp_ref[...] = jnp.broadcast_to(perp, (1, 1))
    used = jnp.sum((cs > EPS).astype(jnp.float32)) / K
    ur_ref[...] = jnp.broadcast_to(used, (1, 1))


def _loss_stats(x, q, cs):
    scalar = jax.ShapeDtypeStruct((1, 1), jnp.float32)
    return pl.pallas_call(
        _loss_stats_body,
        out_shape=[jax.ShapeDtypeStruct((N, D), jnp.float32)] + [scalar] * 5,
    )(x, q, cs)


def kernel(inputs, embedding, cluster_size):
    x = inputs.reshape(N, D)
    sqx = jnp.sum(x ** 2, axis=1, keepdims=True)
    sqe = jnp.sum(embedding ** 2, axis=0, keepdims=True)
    idx, table = _dist_argmin(x, embedding, sqx, sqe)
    quantized = _get_sc_gather()(table, idx)
    qst, el, cl, vl, pp, ur = _loss_stats(x, quantized, cluster_size)
    return (
        qst.reshape(inputs.shape),
        vl[0, 0],
        el[0, 0],
        cl[0, 0],
        pp[0, 0],
        ur[0, 0],
        idx.reshape(inputs.shape[0], -1),
    )
